# baseline (speedup 1.0000x reference)
import functools
import math

import jax
import jax.numpy as jnp
from jax import lax
from jax.experimental import pallas as pl
from jax.experimental.pallas import tpu as pltpu
from jax.experimental.pallas import tpu_sc as plsc

LANES = 16


def _mesh():
    return plsc.VectorSubcoreMesh(core_axis_name="c", subcore_axis_name="s")


def _wid():
    info = plsc.get_sparse_core_info()
    return lax.axis_index("s") * info.num_cores + lax.axis_index("c")


@functools.lru_cache(maxsize=None)
def _build_reformat(V, D):
    info = plsc.get_sparse_core_info()
    NW = info.num_cores * info.num_subcores
    VB = 128
    nblk_full = V // VB
    tail = V - nblk_full * VB
    per_w = (nblk_full + NW - 1) // NW

    @functools.partial(
        pl.kernel,
        mesh=_mesh(),
        out_type=jax.ShapeDtypeStruct((V, VB), jnp.float32),
        scratch_types=[
            pltpu.VMEM((D, VB), jnp.float32),
            pltpu.VMEM((D, VB), jnp.float32),
            pltpu.VMEM((VB, VB), jnp.float32),
            pltpu.VMEM((VB, VB), jnp.float32),
            pltpu.SemaphoreType.DMA,
            pltpu.SemaphoreType.DMA,
            pltpu.SemaphoreType.DMA,
            pltpu.SemaphoreType.DMA,
        ],
        compiler_params=pltpu.CompilerParams(
            use_tc_tiling_on_sc=True, needs_layout_passes=False),
    )
    def k(tt_hbm, tail_hbm, g_hbm,
          in0, in1, tr0, tr1, semr0, semr1, semw0, semw1):
        w = _wid()
        ins = (in0, in1)
        trs = (tr0, tr1)
        semr = (semr0, semr1)
        semw = (semw0, semw1)
        lane = lax.iota(jnp.int32, LANES)

        def vt_of(t):
            return w + t * NW

        def fire_read(t, buf, width):
            vt = vt_of(t)
            for dt in range(D // 8):
                pltpu.async_copy(
                    tt_hbm.at[pl.ds(dt * 8, 8), pl.ds(vt * VB, width)],
                    ins[buf].at[pl.ds(dt * 8, 8), pl.ds(0, width)],
                    semr[buf])

        def drain_read(buf, width):
            for dt in range(D // 8):
                pltpu.make_async_copy(
                    tt_hbm.at[pl.ds(0, 8), pl.ds(0, width)],
                    ins[buf].at[pl.ds(0, 8), pl.ds(0, width)],
                    semr[buf]).wait()

        def transpose(buf, width):
            iv = ins[buf]
            tv = trs[buf]

            def body(d, carry):
                for c in range(width // LANES):
                    v = iv[d, pl.ds(c * LANES, LANES)]
                    row = lane + c * LANES
                    dvec = jnp.full((LANES,), d, jnp.int32)
                    plsc.store_scatter(tv, [row, dvec], v)
                return carry

            lax.fori_loop(0, D, body, 0)

        def fire_write(t, buf, width):
            vt = vt_of(t)
            pltpu.async_copy(
                trs[buf].at[pl.ds(0, width)],
                g_hbm.at[pl.ds(vt * VB, width)], semw[buf])

        def drain_write(buf, width):
            pltpu.make_async_copy(
                trs[buf].at[pl.ds(0, width)],
                g_hbm.at[pl.ds(0, width)], semw[buf]).wait()

        nt_u = nblk_full // NW

        def t_body(t, carry):
            buf = lax.rem(t, 2)

            @pl.when(vt_of(t + 1) < nblk_full)
            def _():
                nb = lax.rem(t + 1, 2)

                @pl.when(nb == 0)
                def _():
                    fire_read(t + 1, 0, VB)

                @pl.when(nb == 1)
                def _():
                    fire_read(t + 1, 1, VB)

            @pl.when(buf == 0)
            def _():
                drain_read(0, VB)

                @pl.when(t >= 2)
                def _():
                    drain_write(0, VB)
                transpose(0, VB)
                fire_write(t, 0, VB)

            @pl.when(buf == 1)
            def _():
                drain_read(1, VB)

                @pl.when(t >= 2)
                def _():
                    drain_write(1, VB)
                transpose(1, VB)
                fire_write(t, 1, VB)
            return carry

        fire_read(0, 0, VB)
        lax.fori_loop(0, nt_u, t_body, 0)
        drain_write(0, VB)
        drain_write(1, VB)

        @pl.when(vt_of(nt_u) < nblk_full)
        def _():
            bufe = lax.rem(nt_u, 2)

            @pl.when(bufe == 0)
            def _():
                drain_read(0, VB)
                transpose(0, VB)
                fire_write(nt_u, 0, VB)
                drain_write(0, VB)

            @pl.when(bufe == 1)
            def _():
                drain_read(1, VB)
                transpose(1, VB)
                fire_write(nt_u, 1, VB)
                drain_write(1, VB)

        if tail:
            @pl.when(w == 0)
            def _():
                pltpu.sync_copy(tail_hbm, tr0.at[pl.ds(0, tail)])
                pltpu.sync_copy(
                    tr0.at[pl.ds(0, tail)],
                    g_hbm.at[pl.ds(nblk_full * VB, tail)])

    return k


@functools.lru_cache(maxsize=None)
def _build_lookup(B, S, D, V):
    info = plsc.get_sparse_core_info()
    NW = info.num_cores * info.num_subcores
    BB = 128
    assert B % BB == 0 and B // BB == NW and S % 8 == 0
    n_s8 = S // 8
    scale = math.sqrt(float(D))
    DL = D // LANES

    @functools.partial(
        pl.kernel,
        mesh=_mesh(),
        out_type=jax.ShapeDtypeStruct((S, D, B), jnp.float32),
        scratch_types=[
            pltpu.VMEM((8, BB), jnp.int32),
            pltpu.VMEM((8, BB), jnp.int32),
            pltpu.VMEM((BB, BB), jnp.float32),
            pltpu.VMEM((BB, BB), jnp.float32),
            pltpu.VMEM((D, BB), jnp.float32),
            pltpu.VMEM((D, BB), jnp.float32),
            pltpu.VMEM((S, D), jnp.float32),
            pltpu.SemaphoreType.DMA,
            pltpu.SemaphoreType.DMA,
            pltpu.SemaphoreType.DMA,
            pltpu.SemaphoreType.DMA,
            pltpu.SemaphoreType.DMA,
        ],
        compiler_params=pltpu.CompilerParams(
            use_tc_tiling_on_sc=True, needs_layout_passes=False),
    )
    def k(xT_hbm, g_hbm, pe_hbm, out_hbm,
          idx0, idx1, rows0, rows1, o0, o1, pe_v,
          semi, semg0, semg1, semw0, semw1):
        w = _wid()
        b0 = w * BB
        pltpu.sync_copy(pe_hbm, pe_v)

        idxs = (idx0, idx1)
        rows = (rows0, rows1)
        outs = (o0, o1)
        semg = (semg0, semg1)
        semw = (semw0, semw1)
        lane = lax.iota(jnp.int32, LANES)

        def fire_gather(ibuf, r, rbuf):
            pltpu.async_copy(
                g_hbm.at[idxs[ibuf].at[r]], rows[rbuf], semg[rbuf])

        def drain_gather(rbuf):
            pltpu.make_async_copy(
                g_hbm.at[pl.ds(0, BB)], rows[rbuf], semg[rbuf]).wait()

        def drain_write(obuf):
            pltpu.make_async_copy(
                outs[obuf],
                out_hbm.at[0, pl.ds(0, D), pl.ds(0, BB)], semw[obuf]).wait()

        def compute(rbuf, obuf, s):
            rv = rows[rbuf]
            ov = outs[obuf]

            def body(b, carry):
                bvec = jnp.full((LANES,), b, jnp.int32)
                for l in range(DL):
                    v = rv[b, pl.ds(l * LANES, LANES)]
                    pe_vec = pe_v[s, pl.ds(l * LANES, LANES)]
                    val = v * scale + pe_vec
                    dvec = lane + (l * LANES)
                    plsc.store_scatter(ov, [dvec, bvec], val)
                return carry

            lax.fori_loop(0, BB, body, 0)

        def fire_write(obuf, s):
            pltpu.async_copy(
                outs[obuf],
                out_hbm.at[s, pl.ds(0, D), pl.ds(b0, BB)], semw[obuf])

        def fire_idx(s8n, ibuf):
            pltpu.async_copy(
                xT_hbm.at[pl.ds(s8n * 8, 8), pl.ds(b0, BB)],
                idxs[ibuf], semi)

        pltpu.sync_copy(xT_hbm.at[pl.ds(0, 8), pl.ds(b0, BB)], idx0)
        fire_gather(0, 0, 0)

        def s8_body(s8, carry):
            cur = lax.rem(s8, 2)
            nxt = lax.rem(s8 + 1, 2)

            @pl.when(s8 + 1 < n_s8)
            def _():
                @pl.when(nxt == 0)
                def _():
                    fire_idx(s8 + 1, 0)

                @pl.when(nxt == 1)
                def _():
                    fire_idx(s8 + 1, 1)

            for r in range(8):
                s = s8 * 8 + r
                A = r % 2
                if r < 7:
                    @pl.when(cur == 0)
                    def _():
                        fire_gather(0, r + 1, 1 - A)

                    @pl.when(cur == 1)
                    def _():
                        fire_gather(1, r + 1, 1 - A)
                else:
                    @pl.when(s8 + 1 < n_s8)
                    def _():
                        pltpu.make_async_copy(
                            xT_hbm.at[pl.ds(0, 8), pl.ds(b0, BB)],
                            idx0, semi).wait()

                        @pl.when(nxt == 0)
                        def _():
                            fire_gather(0, 0, 0)

                        @pl.when(nxt == 1)
                        def _():
                            fire_gather(1, 0, 0)

                drain_gather(A)

                @pl.when(s >= 2)
                def _():
                    drain_write(A)

                compute(A, A, s)
                fire_write(A, s)
            return carry

        lax.fori_loop(0, n_s8, s8_body, 0)
        drain_write(0)
        drain_write(1)

    return k


def kernel(x, table, pe):
    B, S = x.shape
    V, D = table.shape
    ka = _build_reformat(V, D)
    kb = _build_lookup(B, S, D, V)
    tt = table.T
    ntail = V % 128
    tail_pad = jnp.pad(table[V - ntail:, :], ((0, 0), (0, 128 - D)))
    g = ka(tt, tail_pad)
    xT = x.T
    pe_s = pe[0, :S, :]
    o = kb(xT, g, pe_s)
    return o.transpose(2, 0, 1)


# speedup vs baseline: 1.0001x; 1.0001x over previous
import functools
import math

import jax
import jax.numpy as jnp
from jax import lax
from jax.experimental import pallas as pl
from jax.experimental.pallas import tpu as pltpu
from jax.experimental.pallas import tpu_sc as plsc

LANES = 16


def _mesh():
    return plsc.VectorSubcoreMesh(core_axis_name="c", subcore_axis_name="s")


def _wid():
    info = plsc.get_sparse_core_info()
    return lax.axis_index("s") * info.num_cores + lax.axis_index("c")


@functools.lru_cache(maxsize=None)
def _build_reformat(V, D):
    info = plsc.get_sparse_core_info()
    NW = info.num_cores * info.num_subcores
    VB = 128
    nblk_full = V // VB
    tail = V - nblk_full * VB
    per_w = (nblk_full + NW - 1) // NW

    @functools.partial(
        pl.kernel,
        mesh=_mesh(),
        out_type=jax.ShapeDtypeStruct((V, VB), jnp.float32),
        scratch_types=[
            pltpu.VMEM((D, VB), jnp.float32),
            pltpu.VMEM((D, VB), jnp.float32),
            pltpu.VMEM((VB, VB), jnp.float32),
            pltpu.VMEM((VB, VB), jnp.float32),
            pltpu.SemaphoreType.DMA,
            pltpu.SemaphoreType.DMA,
            pltpu.SemaphoreType.DMA,
            pltpu.SemaphoreType.DMA,
        ],
        compiler_params=pltpu.CompilerParams(
            use_tc_tiling_on_sc=True, needs_layout_passes=False),
    )
    def k(tt_hbm, tail_hbm, g_hbm,
          in0, in1, tr0, tr1, semr0, semr1, semw0, semw1):
        w = _wid()
        ins = (in0, in1)
        trs = (tr0, tr1)
        semr = (semr0, semr1)
        semw = (semw0, semw1)
        lane = lax.iota(jnp.int32, LANES)

        def vt_of(t):
            return w + t * NW

        def fire_read(t, buf, width):
            vt = vt_of(t)
            for dt in range(D // 8):
                pltpu.async_copy(
                    tt_hbm.at[pl.ds(dt * 8, 8), pl.ds(vt * VB, width)],
                    ins[buf].at[pl.ds(dt * 8, 8), pl.ds(0, width)],
                    semr[buf])

        def drain_read(buf, width):
            for dt in range(D // 8):
                pltpu.make_async_copy(
                    tt_hbm.at[pl.ds(0, 8), pl.ds(0, width)],
                    ins[buf].at[pl.ds(0, 8), pl.ds(0, width)],
                    semr[buf]).wait()

        def transpose(buf, width):
            iv = ins[buf]
            tv = trs[buf]
            rowvecs = [lane + c * LANES for c in range(width // LANES)]

            def body(dq, dvec0):
                dvec = dvec0
                for u in range(4):
                    d = dq * 4 + u
                    for c in range(width // LANES):
                        v = iv[d, pl.ds(c * LANES, LANES)]
                        plsc.store_scatter(tv, [rowvecs[c], dvec], v)
                    dvec = dvec + 1
                return dvec

            lax.fori_loop(0, D // 4, body, jnp.zeros((LANES,), jnp.int32))

        def fire_write(t, buf, width):
            vt = vt_of(t)
            pltpu.async_copy(
                trs[buf].at[pl.ds(0, width)],
                g_hbm.at[pl.ds(vt * VB, width)], semw[buf])

        def drain_write(buf, width):
            pltpu.make_async_copy(
                trs[buf].at[pl.ds(0, width)],
                g_hbm.at[pl.ds(0, width)], semw[buf]).wait()

        nt_u = nblk_full // NW

        def t_body(t, carry):
            buf = lax.rem(t, 2)

            @pl.when(vt_of(t + 1) < nblk_full)
            def _():
                nb = lax.rem(t + 1, 2)

                @pl.when(nb == 0)
                def _():
                    fire_read(t + 1, 0, VB)

                @pl.when(nb == 1)
                def _():
                    fire_read(t + 1, 1, VB)

            @pl.when(buf == 0)
            def _():
                drain_read(0, VB)

                @pl.when(t >= 2)
                def _():
                    drain_write(0, VB)
                transpose(0, VB)
                fire_write(t, 0, VB)

            @pl.when(buf == 1)
            def _():
                drain_read(1, VB)

                @pl.when(t >= 2)
                def _():
                    drain_write(1, VB)
                transpose(1, VB)
                fire_write(t, 1, VB)
            return carry

        fire_read(0, 0, VB)
        lax.fori_loop(0, nt_u, t_body, 0)
        drain_write(0, VB)
        drain_write(1, VB)

        @pl.when(vt_of(nt_u) < nblk_full)
        def _():
            bufe = lax.rem(nt_u, 2)

            @pl.when(bufe == 0)
            def _():
                drain_read(0, VB)
                transpose(0, VB)
                fire_write(nt_u, 0, VB)
                drain_write(0, VB)

            @pl.when(bufe == 1)
            def _():
                drain_read(1, VB)
                transpose(1, VB)
                fire_write(nt_u, 1, VB)
                drain_write(1, VB)

        if tail:
            @pl.when(w == 0)
            def _():
                pltpu.sync_copy(tail_hbm, tr0.at[pl.ds(0, tail)])
                pltpu.sync_copy(
                    tr0.at[pl.ds(0, tail)],
                    g_hbm.at[pl.ds(nblk_full * VB, tail)])

    return k


@functools.lru_cache(maxsize=None)
def _build_lookup(B, S, D, V):
    info = plsc.get_sparse_core_info()
    NW = info.num_cores * info.num_subcores
    BB = 128
    assert B % BB == 0 and B // BB == NW and S % 8 == 0
    n_s8 = S // 8
    scale = math.sqrt(float(D))
    DL = D // LANES

    @functools.partial(
        pl.kernel,
        mesh=_mesh(),
        out_type=jax.ShapeDtypeStruct((S, D, B), jnp.float32),
        scratch_types=[
            pltpu.VMEM((8, BB), jnp.int32),
            pltpu.VMEM((8, BB), jnp.int32),
            pltpu.VMEM((BB, BB), jnp.float32),
            pltpu.VMEM((BB, BB), jnp.float32),
            pltpu.VMEM((D, BB), jnp.float32),
            pltpu.VMEM((D, BB), jnp.float32),
            pltpu.VMEM((S, D), jnp.float32),
            pltpu.SemaphoreType.DMA,
            pltpu.SemaphoreType.DMA,
            pltpu.SemaphoreType.DMA,
            pltpu.SemaphoreType.DMA,
            pltpu.SemaphoreType.DMA,
        ],
        compiler_params=pltpu.CompilerParams(
            use_tc_tiling_on_sc=True, needs_layout_passes=False),
    )
    def k(xT_hbm, g_hbm, pe_hbm, out_hbm,
          idx0, idx1, rows0, rows1, o0, o1, pe_v,
          semi, semg0, semg1, semw0, semw1):
        w = _wid()
        b0 = w * BB
        pltpu.sync_copy(pe_hbm, pe_v)

        idxs = (idx0, idx1)
        rows = (rows0, rows1)
        outs = (o0, o1)
        semg = (semg0, semg1)
        semw = (semw0, semw1)
        lane = lax.iota(jnp.int32, LANES)

        def fire_gather(ibuf, r, rbuf):
            pltpu.async_copy(
                g_hbm.at[idxs[ibuf].at[r]], rows[rbuf], semg[rbuf])

        def drain_gather(rbuf):
            pltpu.make_async_copy(
                g_hbm.at[pl.ds(0, BB)], rows[rbuf], semg[rbuf]).wait()

        def drain_write(obuf):
            pltpu.make_async_copy(
                outs[obuf],
                out_hbm.at[0, pl.ds(0, D), pl.ds(0, BB)], semw[obuf]).wait()

        def compute(rbuf, obuf, s):
            rv = rows[rbuf]
            ov = outs[obuf]
            pes = [pe_v[s, pl.ds(l * LANES, LANES)] for l in range(DL)]
            dvecs = [lane + l * LANES for l in range(DL)]

            def body(i, bvec0):
                bvec = bvec0
                for u in range(4):
                    b = i * 4 + u
                    for l in range(DL):
                        v = rv[b, pl.ds(l * LANES, LANES)]
                        val = v * scale + pes[l]
                        plsc.store_scatter(ov, [dvecs[l], bvec], val)
                    bvec = bvec + 1
                return bvec

            lax.fori_loop(0, BB // 4, body, jnp.zeros((LANES,), jnp.int32))

        def fire_write(obuf, s):
            pltpu.async_copy(
                outs[obuf],
                out_hbm.at[s, pl.ds(0, D), pl.ds(b0, BB)], semw[obuf])

        def fire_idx(s8n, ibuf):
            pltpu.async_copy(
                xT_hbm.at[pl.ds(s8n * 8, 8), pl.ds(b0, BB)],
                idxs[ibuf], semi)

        pltpu.sync_copy(xT_hbm.at[pl.ds(0, 8), pl.ds(b0, BB)], idx0)
        fire_gather(0, 0, 0)

        def s8_body(s8, carry):
            cur = lax.rem(s8, 2)
            nxt = lax.rem(s8 + 1, 2)

            @pl.when(s8 + 1 < n_s8)
            def _():
                @pl.when(nxt == 0)
                def _():
                    fire_idx(s8 + 1, 0)

                @pl.when(nxt == 1)
                def _():
                    fire_idx(s8 + 1, 1)

            for r in range(8):
                s = s8 * 8 + r
                A = r % 2
                if r < 7:
                    @pl.when(cur == 0)
                    def _():
                        fire_gather(0, r + 1, 1 - A)

                    @pl.when(cur == 1)
                    def _():
                        fire_gather(1, r + 1, 1 - A)
                else:
                    @pl.when(s8 + 1 < n_s8)
                    def _():
                        pltpu.make_async_copy(
                            xT_hbm.at[pl.ds(0, 8), pl.ds(b0, BB)],
                            idx0, semi).wait()

                        @pl.when(nxt == 0)
                        def _():
                            fire_gather(0, 0, 0)

                        @pl.when(nxt == 1)
                        def _():
                            fire_gather(1, 0, 0)

                drain_gather(A)

                @pl.when(s >= 2)
                def _():
                    drain_write(A)

                compute(A, A, s)
                fire_write(A, s)
            return carry

        lax.fori_loop(0, n_s8, s8_body, 0)
        drain_write(0)
        drain_write(1)

    return k


def kernel(x, table, pe):
    B, S = x.shape
    V, D = table.shape
    ka = _build_reformat(V, D)
    kb = _build_lookup(B, S, D, V)
    tt = table.T
    ntail = V % 128
    tail_pad = jnp.pad(table[V - ntail:, :], ((0, 0), (0, 128 - D)))
    g = ka(tt, tail_pad)
    xT = x.T
    pe_s = pe[0, :S, :]
    o = kb(xT, g, pe_s)
    return o.transpose(2, 0, 1)


# speedup vs baseline: 1.8672x; 1.8670x over previous
import functools
import math

import jax
import jax.numpy as jnp
from jax import lax
from jax.experimental import pallas as pl
from jax.experimental.pallas import tpu as pltpu
from jax.experimental.pallas import tpu_sc as plsc

LANES = 16


def _mesh():
    return plsc.VectorSubcoreMesh(core_axis_name="c", subcore_axis_name="s")


def _wid():
    info = plsc.get_sparse_core_info()
    return lax.axis_index("s") * info.num_cores + lax.axis_index("c")


@functools.lru_cache(maxsize=None)
def _build_reformat(V, D):
    info = plsc.get_sparse_core_info()
    NW = info.num_cores * info.num_subcores
    VB = 128
    nblk_full = V // VB
    tail = V - nblk_full * VB
    per_w = (nblk_full + NW - 1) // NW

    @functools.partial(
        pl.kernel,
        mesh=_mesh(),
        out_type=jax.ShapeDtypeStruct((V, VB), jnp.float32),
        scratch_types=[
            pltpu.VMEM((D, VB), jnp.float32),
            pltpu.VMEM((D, VB), jnp.float32),
            pltpu.VMEM((VB, VB), jnp.float32),
            pltpu.VMEM((VB, VB), jnp.float32),
            pltpu.SemaphoreType.DMA,
            pltpu.SemaphoreType.DMA,
            pltpu.SemaphoreType.DMA,
            pltpu.SemaphoreType.DMA,
        ],
        compiler_params=pltpu.CompilerParams(
            use_tc_tiling_on_sc=True, needs_layout_passes=False),
    )
    def k(tt_hbm, tail_hbm, g_hbm,
          in0, in1, tr0, tr1, semr0, semr1, semw0, semw1):
        w = _wid()
        ins = (in0, in1)
        trs = (tr0, tr1)
        semr = (semr0, semr1)
        semw = (semw0, semw1)
        lane = lax.iota(jnp.int32, LANES)

        def vt_of(t):
            return w + t * NW

        def fire_read(t, buf, width):
            vt = vt_of(t)
            for dt in range(D // 8):
                pltpu.async_copy(
                    tt_hbm.at[pl.ds(dt * 8, 8), pl.ds(vt * VB, width)],
                    ins[buf].at[pl.ds(dt * 8, 8), pl.ds(0, width)],
                    semr[buf])

        def drain_read(buf, width):
            for dt in range(D // 8):
                pltpu.make_async_copy(
                    tt_hbm.at[pl.ds(0, 8), pl.ds(0, width)],
                    ins[buf].at[pl.ds(0, 8), pl.ds(0, width)],
                    semr[buf]).wait()

        def transpose(buf, width):
            iv = ins[buf]
            tv = trs[buf]
            rots = [(lane + k) & 15 for k in range(LANES)]

            def body(cb, carry):
                vcol = lane + cb * LANES
                for db in range(D // LANES):
                    d0 = db * LANES
                    for k in range(LANES):
                        rsel = rots[k] + d0
                        t = plsc.load_gather(iv, [rsel, vcol])
                        plsc.store_scatter(tv, [vcol, rsel], t)
                return carry

            lax.fori_loop(0, width // LANES, body, 0)

        def fire_write(t, buf, width):
            vt = vt_of(t)
            pltpu.async_copy(
                trs[buf].at[pl.ds(0, width)],
                g_hbm.at[pl.ds(vt * VB, width)], semw[buf])

        def drain_write(buf, width):
            pltpu.make_async_copy(
                trs[buf].at[pl.ds(0, width)],
                g_hbm.at[pl.ds(0, width)], semw[buf]).wait()

        nt_u = nblk_full // NW

        def t_body(t, carry):
            buf = lax.rem(t, 2)

            @pl.when(vt_of(t + 1) < nblk_full)
            def _():
                nb = lax.rem(t + 1, 2)

                @pl.when(nb == 0)
                def _():
                    fire_read(t + 1, 0, VB)

                @pl.when(nb == 1)
                def _():
                    fire_read(t + 1, 1, VB)

            @pl.when(buf == 0)
            def _():
                drain_read(0, VB)

                @pl.when(t >= 2)
                def _():
                    drain_write(0, VB)
                transpose(0, VB)
                fire_write(t, 0, VB)

            @pl.when(buf == 1)
            def _():
                drain_read(1, VB)

                @pl.when(t >= 2)
                def _():
                    drain_write(1, VB)
                transpose(1, VB)
                fire_write(t, 1, VB)
            return carry

        fire_read(0, 0, VB)
        lax.fori_loop(0, nt_u, t_body, 0)
        drain_write(0, VB)
        drain_write(1, VB)

        @pl.when(vt_of(nt_u) < nblk_full)
        def _():
            bufe = lax.rem(nt_u, 2)

            @pl.when(bufe == 0)
            def _():
                drain_read(0, VB)
                transpose(0, VB)
                fire_write(nt_u, 0, VB)
                drain_write(0, VB)

            @pl.when(bufe == 1)
            def _():
                drain_read(1, VB)
                transpose(1, VB)
                fire_write(nt_u, 1, VB)
                drain_write(1, VB)

        if tail:
            @pl.when(w == 0)
            def _():
                pltpu.sync_copy(tail_hbm, tr0.at[pl.ds(0, tail)])
                pltpu.sync_copy(
                    tr0.at[pl.ds(0, tail)],
                    g_hbm.at[pl.ds(nblk_full * VB, tail)])

    return k


@functools.lru_cache(maxsize=None)
def _build_lookup(B, S, D, V):
    info = plsc.get_sparse_core_info()
    NW = info.num_cores * info.num_subcores
    BB = 128
    assert B % BB == 0 and B // BB == NW and S % 8 == 0
    n_s8 = S // 8
    scale = math.sqrt(float(D))
    DL = D // LANES

    @functools.partial(
        pl.kernel,
        mesh=_mesh(),
        out_type=jax.ShapeDtypeStruct((S, D, B), jnp.float32),
        scratch_types=[
            pltpu.VMEM((8, BB), jnp.int32),
            pltpu.VMEM((8, BB), jnp.int32),
            pltpu.VMEM((BB, BB), jnp.float32),
            pltpu.VMEM((BB, BB), jnp.float32),
            pltpu.VMEM((D, BB), jnp.float32),
            pltpu.VMEM((D, BB), jnp.float32),
            pltpu.VMEM((S, D), jnp.float32),
            pltpu.SemaphoreType.DMA,
            pltpu.SemaphoreType.DMA,
            pltpu.SemaphoreType.DMA,
            pltpu.SemaphoreType.DMA,
            pltpu.SemaphoreType.DMA,
        ],
        compiler_params=pltpu.CompilerParams(
            use_tc_tiling_on_sc=True, needs_layout_passes=False),
    )
    def k(xT_hbm, g_hbm, pe_hbm, out_hbm,
          idx0, idx1, rows0, rows1, o0, o1, pe_v,
          semi, semg0, semg1, semw0, semw1):
        w = _wid()
        b0 = w * BB
        pltpu.sync_copy(pe_hbm, pe_v)

        idxs = (idx0, idx1)
        rows = (rows0, rows1)
        outs = (o0, o1)
        semg = (semg0, semg1)
        semw = (semw0, semw1)
        lane = lax.iota(jnp.int32, LANES)

        def fire_gather(ibuf, r, rbuf):
            pltpu.async_copy(
                g_hbm.at[idxs[ibuf].at[r]], rows[rbuf], semg[rbuf])

        def drain_gather(rbuf):
            pltpu.make_async_copy(
                g_hbm.at[pl.ds(0, BB)], rows[rbuf], semg[rbuf]).wait()

        def drain_write(obuf):
            pltpu.make_async_copy(
                outs[obuf],
                out_hbm.at[0, pl.ds(0, D), pl.ds(0, BB)], semw[obuf]).wait()

        def compute(rbuf, obuf, s):
            rv = rows[rbuf]
            ov = outs[obuf]
            pes = [pe_v[s, pl.ds(l * LANES, LANES)] for l in range(DL)]
            dcols = [lane + l * LANES for l in range(DL)]
            rots = [(lane + k) & 15 for k in range(LANES)]

            def body(bb, carry):
                b0 = bb * LANES
                for db in range(DL):
                    for k in range(LANES):
                        rowv = rots[k] + b0
                        t = plsc.load_gather(rv, [rowv, dcols[db]])
                        val = t * scale + pes[db]
                        plsc.store_scatter(ov, [dcols[db], rowv], val)
                return carry

            lax.fori_loop(0, BB // LANES, body, 0)

        def fire_write(obuf, s):
            pltpu.async_copy(
                outs[obuf],
                out_hbm.at[s, pl.ds(0, D), pl.ds(b0, BB)], semw[obuf])

        def fire_idx(s8n, ibuf):
            pltpu.async_copy(
                xT_hbm.at[pl.ds(s8n * 8, 8), pl.ds(b0, BB)],
                idxs[ibuf], semi)

        pltpu.sync_copy(xT_hbm.at[pl.ds(0, 8), pl.ds(b0, BB)], idx0)
        fire_gather(0, 0, 0)

        def s8_body(s8, carry):
            cur = lax.rem(s8, 2)
            nxt = lax.rem(s8 + 1, 2)

            @pl.when(s8 + 1 < n_s8)
            def _():
                @pl.when(nxt == 0)
                def _():
                    fire_idx(s8 + 1, 0)

                @pl.when(nxt == 1)
                def _():
                    fire_idx(s8 + 1, 1)

            for r in range(8):
                s = s8 * 8 + r
                A = r % 2
                if r < 7:
                    @pl.when(cur == 0)
                    def _():
                        fire_gather(0, r + 1, 1 - A)

                    @pl.when(cur == 1)
                    def _():
                        fire_gather(1, r + 1, 1 - A)
                else:
                    @pl.when(s8 + 1 < n_s8)
                    def _():
                        pltpu.make_async_copy(
                            xT_hbm.at[pl.ds(0, 8), pl.ds(b0, BB)],
                            idx0, semi).wait()

                        @pl.when(nxt == 0)
                        def _():
                            fire_gather(0, 0, 0)

                        @pl.when(nxt == 1)
                        def _():
                            fire_gather(1, 0, 0)

                drain_gather(A)

                @pl.when(s >= 2)
                def _():
                    drain_write(A)

                compute(A, A, s)
                fire_write(A, s)
            return carry

        lax.fori_loop(0, n_s8, s8_body, 0)
        drain_write(0)
        drain_write(1)

    return k


def kernel(x, table, pe):
    B, S = x.shape
    V, D = table.shape
    ka = _build_reformat(V, D)
    kb = _build_lookup(B, S, D, V)
    tt = table.T
    ntail = V % 128
    tail_pad = jnp.pad(table[V - ntail:, :], ((0, 0), (0, 128 - D)))
    g = ka(tt, tail_pad)
    xT = x.T
    pe_s = pe[0, :S, :]
    o = kb(xT, g, pe_s)
    return o.transpose(2, 0, 1)
